# Initial kernel scaffold; baseline (speedup 1.0000x reference)
import functools

import jax
import jax.numpy as jnp
from jax import lax
from jax.experimental import pallas as pl
from jax.experimental.pallas import tpu as pltpu
from jax.experimental.pallas import tpu_sc as plsc

_NCLS = 1000
_B = 16384
_BLK = 512

_NPAD = 1024


def _tc_body(yh_ref, y_ref, loss_ref, cls_ref):
    yh = yh_ref[...]
    yv = y_ref[...]
    m = jnp.max(yh, axis=1, keepdims=True)
    lse = jnp.log(jnp.sum(jnp.exp(yh - m), axis=1, keepdims=True))
    sy = jnp.sum(yv, axis=1)
    syh = jnp.sum(yv * yh, axis=1)
    loss_ref[...] = sy * (m[:, 0] + lse[:, 0]) - syh
    ym = jnp.max(yv, axis=1, keepdims=True)
    colid = lax.broadcasted_iota(jnp.int32, yv.shape, 1)
    cls_ref[...] = jnp.min(jnp.where(yv == ym, colid, _NCLS), axis=1)


def _tc_loss(y_hat, y):
    grid = (_B // _BLK,)
    return pl.pallas_call(
        _tc_body,
        grid=grid,
        in_specs=[
            pl.BlockSpec((_BLK, _NCLS), lambda i: (i, 0)),
            pl.BlockSpec((_BLK, _NCLS), lambda i: (i, 0)),
        ],
        out_specs=[
            pl.BlockSpec((_BLK,), lambda i: (i,)),
            pl.BlockSpec((_BLK,), lambda i: (i,)),
        ],
        out_shape=[
            jax.ShapeDtypeStruct((_B,), jnp.float32),
            jax.ShapeDtypeStruct((_B,), jnp.int32),
        ],
    )(y_hat, y)


def _sc_body(loss_hbm, cls_hbm, out_hbm,
             loss_v, cls_v, acc, idx_v, tmp_s, tmp_c, out_v, shared):
    c = lax.axis_index("c")
    s = lax.axis_index("s")

    @pl.when(c == 0)
    def _():
        def zero_row(i, _):
            acc[i] = jnp.zeros((16,), jnp.float32)
            return 0
        lax.fori_loop(0, 128, zero_row, 0)

        @pl.when(s == 0)
        def _():
            pltpu.sync_copy(acc, shared)
        plsc.subcore_barrier()

        def idx_row(i, _):
            idx_v[pl.ds(i * 16, 16)] = lax.iota(jnp.int32, 16) + i * 16
            return 0
        lax.fori_loop(0, 8, idx_row, 0)

        n_per = _B // 16
        base = s * n_per
        pltpu.sync_copy(loss_hbm.at[pl.ds(base, n_per)], loss_v)
        pltpu.sync_copy(cls_hbm.at[pl.ds(base, n_per)], cls_v)

        ones = jnp.ones((16,), jnp.float32)

        def accum(j, _):
            lv = loss_v[pl.ds(j * 16, 16)]
            cv = cls_v[pl.ds(j * 16, 16)]
            row = lax.shift_right_logical(cv, 4)
            col = jnp.bitwise_and(cv, 15)
            plsc.addupdate_scatter(acc, [row, col], lv)
            plsc.addupdate_scatter(acc, [row + 64, col], ones)
            return 0
        lax.fori_loop(0, n_per // 16, accum, 0)

        pltpu.sync_copy(acc, shared.at[idx_v], add=True)
        plsc.subcore_barrier()

        r = s * 4
        pltpu.sync_copy(shared.at[pl.ds(r, 4)], tmp_s)
        pltpu.sync_copy(shared.at[pl.ds(64 + r, 4)], tmp_c)

        def div_row(j, _):
            out_v[j] = tmp_s[j] / tmp_c[j]
            return 0
        lax.fori_loop(0, 4, div_row, 0)
        pltpu.sync_copy(out_v, out_hbm.at[pl.ds(r, 4)])


def _sc_segment_mean(loss, cls):
    mesh = plsc.VectorSubcoreMesh(core_axis_name="c", subcore_axis_name="s")
    n_per = _B // 16
    f = functools.partial(
        pl.kernel,
        mesh=mesh,
        out_type=jax.ShapeDtypeStruct((64, 16), jnp.float32),
        scratch_types=[
            pltpu.VMEM((n_per,), jnp.float32),
            pltpu.VMEM((n_per,), jnp.int32),
            pltpu.VMEM((128, 16), jnp.float32),
            pltpu.VMEM((128,), jnp.int32),
            pltpu.VMEM((4, 16), jnp.float32),
            pltpu.VMEM((4, 16), jnp.float32),
            pltpu.VMEM((4, 16), jnp.float32),
            pltpu.VMEM_SHARED((128, 16), jnp.float32),
        ],
    )(_sc_body)
    return f(loss, cls)


def kernel(y_hat, y):
    loss, cls = _tc_loss(y_hat, y)
    out = _sc_segment_mean(loss, cls)
    return out.reshape(_NPAD)[:_NCLS]


# speedup vs baseline: 1.0674x; 1.0674x over previous
import functools

import jax
import jax.numpy as jnp
from jax import lax
from jax.experimental import pallas as pl
from jax.experimental.pallas import tpu as pltpu
from jax.experimental.pallas import tpu_sc as plsc

_NCLS = 1000
_B = 16384
_BLK = 512

_NPAD = 1024


def _tc_body(yh_ref, y_ref, loss_ref, cls_ref):
    yh = yh_ref[...]
    yv = y_ref[...]
    m = jnp.max(yh, axis=1, keepdims=True)
    lse = jnp.log(jnp.sum(jnp.exp(yh - m), axis=1, keepdims=True))
    sy = jnp.sum(yv, axis=1)
    syh = jnp.sum(yv * yh, axis=1)
    loss_ref[...] = sy * (m[:, 0] + lse[:, 0]) - syh
    ym = jnp.max(yv, axis=1, keepdims=True)
    colid = lax.broadcasted_iota(jnp.int32, yv.shape, 1)
    cls_ref[...] = jnp.min(jnp.where(yv == ym, colid, _NCLS), axis=1)


def _tc_loss(y_hat, y):
    grid = (_B // _BLK,)
    return pl.pallas_call(
        _tc_body,
        grid=grid,
        in_specs=[
            pl.BlockSpec((_BLK, _NCLS), lambda i: (i, 0)),
            pl.BlockSpec((_BLK, _NCLS), lambda i: (i, 0)),
        ],
        out_specs=[
            pl.BlockSpec((_BLK,), lambda i: (i,)),
            pl.BlockSpec((_BLK,), lambda i: (i,)),
        ],
        out_shape=[
            jax.ShapeDtypeStruct((_B,), jnp.float32),
            jax.ShapeDtypeStruct((_B,), jnp.int32),
        ],
    )(y_hat, y)


def _sc_body(loss_hbm, cls_hbm, out_hbm,
             loss_v, cls_v, acc, big_v, out_v, shared):
    c = lax.axis_index("c")
    s = lax.axis_index("s")

    @pl.when(c == 0)
    def _():
        def zero_chunk(i, _):
            acc[pl.ds(i * 16, 16)] = jnp.zeros((16,), jnp.float32)
            return 0
        lax.fori_loop(0, 2 * _NPAD // 16, zero_chunk, 0)

        n_per = _B // 16
        base = s * n_per
        pltpu.sync_copy(loss_hbm.at[pl.ds(base, n_per)], loss_v)
        pltpu.sync_copy(cls_hbm.at[pl.ds(base, n_per)], cls_v)

        ones = jnp.ones((16,), jnp.float32)

        def accum(j, _):
            lv = loss_v[pl.ds(j * 16, 16)]
            cv = cls_v[pl.ds(j * 16, 16)]
            plsc.addupdate_scatter(acc, [cv], lv)
            plsc.addupdate_scatter(acc, [cv + _NPAD], ones)
            return 0
        lax.fori_loop(0, n_per // 16, accum, 0)

        pltpu.sync_copy(acc, shared.at[s])
        plsc.subcore_barrier()
        pltpu.sync_copy(shared, big_v)

        cbase = s * 64
        for k in range(4):
            def red(t, v):
                vs, vc = v
                vs = vs + big_v[t, pl.ds(cbase + k * 16, 16)]
                vc = vc + big_v[t, pl.ds(_NPAD + cbase + k * 16, 16)]
                return (vs, vc)
            z = jnp.zeros((16,), jnp.float32)
            vs, vc = lax.fori_loop(0, 16, red, (z, z))
            out_v[pl.ds(k * 16, 16)] = vs / vc
        pltpu.sync_copy(out_v, out_hbm.at[pl.ds(cbase, 64)])


def _sc_segment_mean(loss, cls):
    mesh = plsc.VectorSubcoreMesh(core_axis_name="c", subcore_axis_name="s")
    n_per = _B // 16
    f = functools.partial(
        pl.kernel,
        mesh=mesh,
        out_type=jax.ShapeDtypeStruct((_NPAD,), jnp.float32),
        compiler_params=pltpu.CompilerParams(needs_layout_passes=False),
        scratch_types=[
            pltpu.VMEM((n_per,), jnp.float32),
            pltpu.VMEM((n_per,), jnp.int32),
            pltpu.VMEM((2 * _NPAD,), jnp.float32),
            pltpu.VMEM((16, 2 * _NPAD), jnp.float32),
            pltpu.VMEM((64,), jnp.float32),
            pltpu.VMEM_SHARED((16, 2 * _NPAD), jnp.float32),
        ],
    )(_sc_body)
    return f(loss, cls)


def kernel(y_hat, y):
    loss, cls = _tc_loss(y_hat, y)
    out = _sc_segment_mean(loss, cls)
    return out[:_NCLS]


# speedup vs baseline: 1.1095x; 1.0395x over previous
import functools

import jax
import jax.numpy as jnp
from jax import lax
from jax.experimental import pallas as pl
from jax.experimental.pallas import tpu as pltpu
from jax.experimental.pallas import tpu_sc as plsc

_NCLS = 1000
_B = 16384
_BLK = 1024

_NPAD = 1024


def _tc_body(yh_ref, y_ref, loss_ref, cls_ref):
    yh = yh_ref[...]
    yv = y_ref[...]
    m = jnp.max(yh, axis=1, keepdims=True)
    lse = jnp.log(jnp.sum(jnp.exp(yh - m), axis=1, keepdims=True))
    sy = jnp.sum(yv, axis=1)
    syh = jnp.sum(yv * yh, axis=1)
    loss_ref[...] = sy * (m[:, 0] + lse[:, 0]) - syh
    ym = jnp.max(yv, axis=1, keepdims=True)
    colid = lax.broadcasted_iota(jnp.int32, yv.shape, 1)
    cls_ref[...] = jnp.min(jnp.where(yv == ym, colid, _NCLS), axis=1)


def _tc_loss(y_hat, y):
    grid = (_B // _BLK,)
    return pl.pallas_call(
        _tc_body,
        grid=grid,
        in_specs=[
            pl.BlockSpec((_BLK, _NCLS), lambda i: (i, 0)),
            pl.BlockSpec((_BLK, _NCLS), lambda i: (i, 0)),
        ],
        out_specs=[
            pl.BlockSpec((_BLK,), lambda i: (i,)),
            pl.BlockSpec((_BLK,), lambda i: (i,)),
        ],
        out_shape=[
            jax.ShapeDtypeStruct((_B,), jnp.float32),
            jax.ShapeDtypeStruct((_B,), jnp.int32),
        ],
    )(y_hat, y)


def _sc_body(loss_hbm, cls_hbm, out_hbm,
             loss_v, cls_v, acc, big_v, out_v, shared):
    c = lax.axis_index("c")
    s = lax.axis_index("s")

    @pl.when(c == 0)
    def _():
        def zero_chunk(i, _):
            acc[pl.ds(i * 16, 16)] = jnp.zeros((16,), jnp.float32)
            return 0
        lax.fori_loop(0, 2 * _NPAD // 16, zero_chunk, 0)

        n_per = _B // 16
        base = s * n_per
        pltpu.sync_copy(loss_hbm.at[pl.ds(base, n_per)], loss_v)
        pltpu.sync_copy(cls_hbm.at[pl.ds(base, n_per)], cls_v)

        ones = jnp.ones((16,), jnp.float32)

        def accum(j, _):
            lv = loss_v[pl.ds(j * 16, 16)]
            cv = cls_v[pl.ds(j * 16, 16)]
            plsc.addupdate_scatter(acc, [cv], lv)
            plsc.addupdate_scatter(acc, [cv + _NPAD], ones)
            return 0
        lax.fori_loop(0, n_per // 16, accum, 0)

        pltpu.sync_copy(acc, shared.at[s])
        plsc.subcore_barrier()
        pltpu.sync_copy(shared, big_v)

        cbase = s * 64
        for k in range(4):
            def red(t, v):
                vs, vc = v
                vs = vs + big_v[t, pl.ds(cbase + k * 16, 16)]
                vc = vc + big_v[t, pl.ds(_NPAD + cbase + k * 16, 16)]
                return (vs, vc)
            z = jnp.zeros((16,), jnp.float32)
            vs, vc = lax.fori_loop(0, 16, red, (z, z))
            out_v[pl.ds(k * 16, 16)] = vs / vc
        pltpu.sync_copy(out_v, out_hbm.at[pl.ds(cbase, 64)])


def _sc_segment_mean(loss, cls):
    mesh = plsc.VectorSubcoreMesh(core_axis_name="c", subcore_axis_name="s")
    n_per = _B // 16
    f = functools.partial(
        pl.kernel,
        mesh=mesh,
        out_type=jax.ShapeDtypeStruct((_NPAD,), jnp.float32),
        compiler_params=pltpu.CompilerParams(needs_layout_passes=False),
        scratch_types=[
            pltpu.VMEM((n_per,), jnp.float32),
            pltpu.VMEM((n_per,), jnp.int32),
            pltpu.VMEM((2 * _NPAD,), jnp.float32),
            pltpu.VMEM((16, 2 * _NPAD), jnp.float32),
            pltpu.VMEM((64,), jnp.float32),
            pltpu.VMEM_SHARED((16, 2 * _NPAD), jnp.float32),
        ],
    )(_sc_body)
    return f(loss, cls)


def kernel(y_hat, y):
    loss, cls = _tc_loss(y_hat, y)
    out = _sc_segment_mean(loss, cls)
    return out[:_NCLS]
